# Initial kernel scaffold; baseline (speedup 1.0000x reference)
import jax
import jax.numpy as jnp
from jax.experimental import pallas as pl


def kernel(ori_feats, latent_feats, labels, r_negative):
    raise NotImplementedError("write your pallas kernel here")


# speedup vs baseline: 112.1375x; 112.1375x over previous
import jax
import jax.numpy as jnp
from jax.experimental import pallas as pl
from jax.experimental.pallas import tpu as pltpu

_TEMP = 0.07
_BLK = 256
_LO = -1071644674
_BIG = 1068498945


def _sortable(x):
    i = jax.lax.bitcast_convert_type(x, jnp.int32)
    return jnp.where(i >= 0, i, i ^ jnp.int32(0x7FFFFFFF))


def _body(r_ref, ori_ref, lat_ref, labc_ref, labr_ref, out_ref,
          on_ref, ln_ref, sq_ref):
    i = pl.program_id(0)
    nblk = pl.num_programs(0)
    n = ori_ref.shape[0]
    blk = labc_ref.shape[0]

    @pl.when(i == 0)
    def _init():
        o = ori_ref[...]
        on = o / jnp.maximum(jnp.sqrt(jnp.sum(o * o, axis=1, keepdims=True)),
                             1e-12)
        on_ref[...] = on
        la = lat_ref[...]
        ln_ref[...] = la / jnp.maximum(
            jnp.sqrt(jnp.sum(la * la, axis=1, keepdims=True)), 1e-12)
        sq_ref[...] = -0.5 * jnp.sum(on * on, axis=1, keepdims=True)
        out_ref[0, 0] = 0.0

    onb = on_ref[pl.ds(i * blk, blk), :]
    lnb = ln_ref[pl.ds(i * blk, blk), :]

    dims = (((1,), (1,)), ((), ()))
    hi_p = jax.lax.Precision.HIGHEST
    key_f = jax.lax.dot_general(onb, on_ref[...], dims,
                                preferred_element_type=jnp.float32,
                                precision=hi_p)
    key_f = key_f + jax.lax.dot_general(
        jnp.ones((blk, 1), jnp.float32), sq_ref[...], dims,
        preferred_element_type=jnp.float32, precision=hi_p)
    slat = jax.lax.dot_general(onb, ln_ref[...], dims,
                               preferred_element_type=jnp.float32,
                               precision=hi_p)
    logit = jnp.exp(slat / _TEMP)

    labc = labc_ref[...]
    labr = labr_ref[...]
    posm = labc == labr
    negcnt = n - jnp.sum(posm.astype(jnp.int32), axis=1, keepdims=True)
    r = r_ref[0, 0]
    k = (r * negcnt.astype(jnp.float32)).astype(jnp.int32)

    keyi = _sortable(jnp.clip(key_f, -1.75, 1.25))
    mk = jnp.where(posm, jnp.int32(_BIG), keyi)

    def bis(_, carry):
        lo, hi = carry
        mid = lo + (hi - lo) // 2
        cnt = jnp.sum((mk <= mid).astype(jnp.int32), axis=1, keepdims=True)
        pred = cnt >= k
        return jnp.where(pred, lo, mid + 1), jnp.where(pred, mid, hi)

    lo0 = jnp.full((blk, 1), _LO, jnp.int32)
    hi0 = jnp.full((blk, 1), _BIG, jnp.int32)
    _, tsel = jax.lax.fori_loop(0, 31, bis, (lo0, hi0))

    lt = mk < tsel
    eq = mk == tsel
    cnt_lt = jnp.sum(lt.astype(jnp.int32), axis=1, keepdims=True)
    col = jax.lax.broadcasted_iota(jnp.int32, (blk, n), 1)

    def bis2(_, carry):
        lo, hi = carry
        mid = lo + (hi - lo) // 2
        cnt = cnt_lt + jnp.sum((eq & (col <= mid)).astype(jnp.int32),
                               axis=1, keepdims=True)
        pred = cnt >= k
        return jnp.where(pred, lo, mid + 1), jnp.where(pred, mid, hi)

    lo20 = jnp.full((blk, 1), -1, jnp.int32)
    hi20 = jnp.full((blk, 1), n - 1, jnp.int32)
    _, jt = jax.lax.fori_loop(0, 13, bis2, (lo20, hi20))

    select = lt | (eq & (col <= jt))
    neg = jnp.sum(jnp.where(select, logit, 0.0), axis=1, keepdims=True)
    pos = jnp.exp(jnp.sum(onb * lnb, axis=1, keepdims=True) / _TEMP)
    bsum = jnp.sum(-jnp.log(pos / (pos + neg)))

    acc = out_ref[0, 0] + bsum
    out_ref[0, 0] = jnp.where(i == nblk - 1, acc / n, acc)


def kernel(ori_feats, latent_feats, labels, r_negative=0.1):
    n, _ = ori_feats.shape
    blk = min(_BLK, n)
    r2 = jnp.asarray(r_negative, jnp.float32).reshape(1, 1)
    labc = labels.astype(jnp.int32).reshape(n, 1)
    labr = labels.astype(jnp.int32).reshape(1, n)
    out = pl.pallas_call(
        _body,
        grid=(n // blk,),
        in_specs=[
            pl.BlockSpec(memory_space=pltpu.SMEM),
            pl.BlockSpec((n, ori_feats.shape[1]), lambda i: (0, 0)),
            pl.BlockSpec((n, latent_feats.shape[1]), lambda i: (0, 0)),
            pl.BlockSpec((blk, 1), lambda i: (i, 0)),
            pl.BlockSpec((1, n), lambda i: (0, 0)),
        ],
        out_specs=pl.BlockSpec(memory_space=pltpu.SMEM),
        out_shape=jax.ShapeDtypeStruct((1, 1), jnp.float32),
        scratch_shapes=[
            pltpu.VMEM((n, ori_feats.shape[1]), jnp.float32),
            pltpu.VMEM((n, latent_feats.shape[1]), jnp.float32),
            pltpu.VMEM((n, 1), jnp.float32),
        ],
    )(r2, ori_feats, latent_feats, labc, labr)
    return out.reshape(())
